# baseline (speedup 1.0000x reference)
import functools

import jax
import jax.numpy as jnp
from jax import lax
from jax.experimental import pallas as pl
from jax.experimental.pallas import tpu as pltpu
from jax.experimental.pallas import tpu_sc as plsc

VOCAB = 100000
EMBED_DIM = 128
BATCH = 4096
SEQ = 200

NC = 2
NS = 16
NW = NC * NS

TOTAL = BATCH * SEQ
B_PER_W = TOTAL // NW
GATHER = 128
K = 2
GROUP = K * GATHER
N_GATHERS = B_PER_W // GATHER
N_GROUPS = B_PER_W // GROUP
NBUF = 3


def _build_kernel():
  mesh = plsc.VectorSubcoreMesh(
      core_axis_name="c", subcore_axis_name="s",
      num_cores=NC, num_subcores=NS)

  @functools.partial(
      pl.kernel,
      out_type=jax.ShapeDtypeStruct((TOTAL, EMBED_DIM), jnp.float32),
      mesh=mesh,
      scratch_types=[
          pltpu.VMEM((N_GATHERS, GATHER), jnp.int32),
          pltpu.VMEM((NBUF, GROUP, EMBED_DIM), jnp.float32),
          pltpu.SemaphoreType.DMA,
          pltpu.SemaphoreType.DMA,
      ],
  )
  def k(idx_hbm, table_hbm, out_hbm, idx_v, rows_v, gsem, osem):
    wid = lax.axis_index("s") * NC + lax.axis_index("c")
    gather_base = wid * N_GATHERS

    pltpu.sync_copy(idx_hbm.at[pl.ds(gather_base, N_GATHERS)], idx_v)

    def gather_copy(g):
      b = lax.rem(lax.div(g, K), NBUF)
      h = lax.rem(g, K)
      return pltpu.make_async_copy(
          table_hbm.at[idx_v.at[g]],
          rows_v.at[b].at[pl.ds(h * GATHER, GATHER)], gsem)

    def write_copy(j):
      b = lax.rem(j, NBUF)
      return pltpu.make_async_copy(
          rows_v.at[b],
          out_hbm.at[pl.ds((wid * N_GROUPS + j) * GROUP, GROUP)], osem)

    for g in range(K):
      gather_copy(g).start()

    def body(j, _):
      @pl.when(j + 1 < N_GROUPS)
      def _():
        @pl.when(j >= NBUF - 1)
        def _():
          write_copy(j - (NBUF - 1)).wait()
        for h in range(K):
          gather_copy((j + 1) * K + h).start()
      for h in range(K):
        gather_copy(j * K + h).wait()
      write_copy(j).start()
      return 0

    lax.fori_loop(0, N_GROUPS, body, 0, unroll=2)
    for j in range(N_GROUPS - NBUF, N_GROUPS):
      write_copy(j).wait()

  return k


_kernel = _build_kernel()


@jax.jit
def kernel(x, table):
  idx = x.astype(jnp.int32).reshape(TOTAL // GATHER, GATHER)
  out = _kernel(idx, table)
  return out.reshape(BATCH, SEQ, EMBED_DIM)
